# baseline (speedup 1.0000x reference)
import functools

import jax
import jax.numpy as jnp
from jax import lax
from jax.experimental import pallas as pl
from jax.experimental.pallas import tpu as pltpu

B = 4
N = 1024
HID = 256
EMB = 128
NC = 2
TM = 256
R = N // TM

_DN = (((1,), (0,)), ((), ()))
_DT = (((0,), (0,)), ((), ()))
_BF = jnp.bfloat16


def _col_of(row_vec, c0):
    seg = row_vec[0:1, c0:c0 + TM]
    m = jnp.broadcast_to(seg, (TM, TM))
    ii = lax.broadcasted_iota(jnp.int32, (TM, TM), 0)
    jj = lax.broadcasted_iota(jnp.int32, (TM, TM), 1)
    return jnp.sum(jnp.where(ii == jj, m, 0.0), axis=1, keepdims=True)


def _stream_body(head, *refs):
    if head:
        (A, W1b, b1, W2b, b2, femb, hW1b, hb1, hW2b, hb2, out_ref,
         ab2, t3, S3, S2, deg, dcol2, emb_s) = refs
    else:
        (A, W1b, b1, W2b, b2, out_ref,
         ab2, t3, S3, S2, deg, dcol2) = refs
    gg = pl.program_id(0)
    q = pl.program_id(1)
    r = pl.program_id(2)
    rbase = r * TM
    f32 = jnp.float32
    cur = gg % 2
    prv = 1 - cur

    @pl.when(q == 0)
    def _q0():
        abp = ab2[pl.ds(prv, 1)][0]
        tj = t3[r]
        S3[r] = lax.dot_general(abp, tj, _DT,
                                preferred_element_type=f32)

        a = A[0]
        w = jnp.where(a > 1e-6, a, 0.0)
        ii = lax.broadcasted_iota(jnp.int32, (TM, N), 0) + rbase
        jj = lax.broadcasted_iota(jnp.int32, (TM, N), 1)
        a_hat = jnp.where(ii == jj, 1.0, w)
        colsum = jnp.sum(a_hat, axis=0, keepdims=True)
        deg[...] = jnp.where(r == 0, colsum, deg[...] + colsum)
        ab2[pl.ds(cur, 1), pl.ds(rbase, TM), :] = a_hat.astype(_BF)[None]

        @pl.when(r == R - 1)
        def _dbuild():
            for ci in range(R):
                dcol2[pl.ds(cur, 1), ci * TM:(ci + 1) * TM, :] = (
                    lax.rsqrt(_col_of(deg[...], ci * TM))[None])

    @pl.when(q == 1)
    def _q1():
        dp = dcol2[pl.ds(prv, 1), pl.ds(rbase, TM), :][0]
        S_rt = jnp.concatenate(
            [S3[j, pl.ds(rbase, TM), :] for j in range(R)], axis=1)
        Z = (dp * S_rt).astype(_BF)
        M = lax.dot_general(Z, W1b[...], _DN, preferred_element_type=f32)
        h = jnp.maximum(M + b1[...], 0.0)
        t2v = (dp * h).astype(_BF)
        abp_rt = ab2[pl.ds(prv, 1), pl.ds(rbase, TM), :][0]
        c2 = lax.dot_general(abp_rt, t2v, _DT,
                             preferred_element_type=f32)
        S2[...] = jnp.where(r == 0, c2, S2[...] + c2)

        a = A[0]
        dc = dcol2[pl.ds(cur, 1), pl.ds(rbase, TM), :][0]
        tt = (dc * a).astype(_BF)
        for j in range(R):
            t3[j, pl.ds(rbase, TM), :] = tt[:, j * TM:(j + 1) * TM]

        @pl.when(r == R - 1)
        def _epilogue():
            acc = jnp.zeros((1, EMB), f32)
            for ci in range(R):
                sl = slice(ci * TM, (ci + 1) * TM)
                dpc = dcol2[pl.ds(prv, 1), sl, :][0]
                Z2 = (dpc * S2[sl, :]).astype(_BF)
                h2 = lax.dot_general(Z2, W2b[...], _DN,
                                     preferred_element_type=f32) + b2[...]
                acc = acc + jnp.sum(h2, axis=0, keepdims=True)
            e = acc * (1.0 / N)
            if head:
                emb_s[pl.ds(jnp.maximum(gg, 1) - 1, 1), :] = e

                @pl.when(gg == B)
                def _head():
                    feat = jnp.concatenate([femb[...], emb_s[...]], axis=1)
                    hh = jnp.maximum(
                        lax.dot_general(feat.astype(_BF), hW1b[...], _DN,
                                        preferred_element_type=f32)
                        + hb1[...], 0.0)
                    out_ref[...] = lax.dot_general(
                        hh.astype(_BF), hW2b[...], _DN,
                        preferred_element_type=f32) + hb2[...]
            else:
                out_ref[0] = e


def _const(*_):
    return (0, 0)


def _adj_map(gg, q, r):
    return (jnp.minimum(gg, B - 1), jnp.where(gg < B, r, R - 1), 0)


def _stream_call(adj, W1, b1, W2, b2, head_args=None):
    f32 = jnp.float32
    head = head_args is not None
    in_specs = [
        pl.BlockSpec((1, TM, N), _adj_map),
        pl.BlockSpec((N, HID), _const),
        pl.BlockSpec((1, HID), _const),
        pl.BlockSpec((HID, EMB), _const),
        pl.BlockSpec((1, EMB), _const),
    ]
    scratch = [
        pltpu.VMEM((2, N, N), _BF),
        pltpu.VMEM((R, N, TM), _BF),
        pltpu.VMEM((R, N, TM), f32),
        pltpu.VMEM((N, HID), f32),
        pltpu.VMEM((1, N), f32),
        pltpu.VMEM((2, N, 1), f32),
    ]
    operands = [adj, W1.astype(_BF), b1, W2.astype(_BF), b2]
    if head:
        in_specs += [
            pl.BlockSpec((B, EMB), _const),
            pl.BlockSpec((2 * EMB, HID), _const),
            pl.BlockSpec((1, HID), _const),
            pl.BlockSpec((HID, NC), _const),
            pl.BlockSpec((1, NC), _const),
        ]
        scratch.append(pltpu.VMEM((B, EMB), f32))
        fe, hW1, hb1, hW2, hb2 = head_args
        operands += [fe, hW1.astype(_BF), hb1, hW2.astype(_BF), hb2]
        out_spec = pl.BlockSpec((B, NC), _const)
        out_shape = jax.ShapeDtypeStruct((B, NC), f32)
    else:
        out_spec = pl.BlockSpec(
            (1, 1, EMB), lambda gg, q, r: (jnp.maximum(gg, 1) - 1, 0, 0))
        out_shape = jax.ShapeDtypeStruct((B, 1, EMB), f32)

    return pl.pallas_call(
        functools.partial(_stream_body, head),
        grid=(B + 1, 2, R),
        in_specs=in_specs,
        out_specs=out_spec,
        out_shape=out_shape,
        scratch_shapes=scratch,
        compiler_params=pltpu.CompilerParams(
            dimension_semantics=("arbitrary", "arbitrary", "arbitrary")),
    )(*operands)


@jax.jit
def kernel(fc_adj, sc_adj, fc_W1, fc_b1, fc_W2, fc_b2,
           sc_W1, sc_b1, sc_W2, sc_b2, h_W1, h_b1, h_W2, h_b2):
    femb = _stream_call(fc_adj, fc_W1, fc_b1.reshape(1, HID),
                        fc_W2, fc_b2.reshape(1, EMB)).reshape(B, EMB)
    out = _stream_call(sc_adj, sc_W1, sc_b1.reshape(1, HID),
                       sc_W2, sc_b2.reshape(1, EMB),
                       head_args=(femb, h_W1, h_b1.reshape(1, HID),
                                  h_W2, h_b2.reshape(1, NC)))
    return out


# speedup vs baseline: 1.1089x; 1.1089x over previous
import functools

import jax
import jax.numpy as jnp
from jax import lax
from jax.experimental import pallas as pl
from jax.experimental.pallas import tpu as pltpu

B = 4
N = 1024
HID = 256
EMB = 128
NC = 2
TM = 256
R = N // TM

_DN = (((1,), (0,)), ((), ()))
_DT = (((0,), (0,)), ((), ()))
_BF = jnp.bfloat16


def _col_of(row_vec, c0):
    seg = row_vec[0:1, c0:c0 + TM]
    m = jnp.broadcast_to(seg, (TM, TM))
    ii = lax.broadcasted_iota(jnp.int32, (TM, TM), 0)
    jj = lax.broadcasted_iota(jnp.int32, (TM, TM), 1)
    return jnp.sum(jnp.where(ii == jj, m, 0.0), axis=1, keepdims=True)


def _stream_body(head, *refs):
    if head:
        (A, W1b, b1, W2b, b2, femb, hW1b, hb1, hW2b, hb2, out_ref,
         ab2, t3, S3, S2, deg, dcol2, emb_s) = refs
    else:
        (A, W1b, b1, W2b, b2, out_ref,
         ab2, t3, S3, S2, deg, dcol2) = refs
    gg = pl.program_id(0)
    q = pl.program_id(1)
    r = pl.program_id(2)
    rbase = r * TM
    f32 = jnp.float32
    cur = gg % 2
    prv = 1 - cur

    @pl.when(q == 0)
    def _q0():
        @pl.when(gg < B)
        def _prep():
            a = A[0]
            w = jnp.where(a > 1e-6, a, 0.0)
            ii = lax.broadcasted_iota(jnp.int32, (TM, N), 0) + rbase
            jj = lax.broadcasted_iota(jnp.int32, (TM, N), 1)
            a_hat = jnp.where(ii == jj, 1.0, w)
            colsum = jnp.sum(a_hat, axis=0, keepdims=True)
            deg[...] = jnp.where(r == 0, colsum, deg[...] + colsum)
            ab2[pl.ds(cur, 1), pl.ds(rbase, TM), :] = a_hat.astype(_BF)[None]

        @pl.when(gg >= 1)
        def _s_dot():
            abp = ab2[pl.ds(prv, 1)][0]
            tj = t3[r]
            S3[r] = lax.dot_general(abp, tj, _DT,
                                    preferred_element_type=f32)

    @pl.when(q == 1)
    def _q1():
        @pl.when(gg < B)
        def _tbuild():
            @pl.when(r == 0)
            def _():
                for ci in range(R):
                    dcol2[pl.ds(cur, 1), ci * TM:(ci + 1) * TM, :] = (
                        lax.rsqrt(_col_of(deg[...], ci * TM))[None])

            a = A[0]
            dc = dcol2[pl.ds(cur, 1), pl.ds(rbase, TM), :][0]
            tt = (dc * a).astype(_BF)
            for j in range(R):
                t3[j, pl.ds(rbase, TM), :] = tt[:, j * TM:(j + 1) * TM]

        @pl.when(gg >= 1)
        def _consume():
            dp = dcol2[pl.ds(prv, 1), pl.ds(rbase, TM), :][0]
            S_rt = jnp.concatenate(
                [S3[j, pl.ds(rbase, TM), :] for j in range(R)], axis=1)
            Z = (dp * S_rt).astype(_BF)
            M = lax.dot_general(Z, W1b[...], _DN, preferred_element_type=f32)
            h = jnp.maximum(M + b1[...], 0.0)
            t2v = (dp * h).astype(_BF)
            abp_rt = ab2[pl.ds(prv, 1), pl.ds(rbase, TM), :][0]
            c2 = lax.dot_general(abp_rt, t2v, _DT,
                                 preferred_element_type=f32)
            S2[...] = jnp.where(r == 0, c2, S2[...] + c2)

            @pl.when(r == R - 1)
            def _epilogue():
                acc = jnp.zeros((1, EMB), f32)
                for ci in range(R):
                    sl = slice(ci * TM, (ci + 1) * TM)
                    dpc = dcol2[pl.ds(prv, 1), sl, :][0]
                    Z2 = (dpc * S2[sl, :]).astype(_BF)
                    h2 = lax.dot_general(Z2, W2b[...], _DN,
                                         preferred_element_type=f32) + b2[...]
                    acc = acc + jnp.sum(h2, axis=0, keepdims=True)
                e = acc * (1.0 / N)
                if head:
                    emb_s[pl.ds(jnp.maximum(gg, 1) - 1, 1), :] = e

                    @pl.when(gg == B)
                    def _head():
                        feat = jnp.concatenate([femb[...], emb_s[...]], axis=1)
                        hh = jnp.maximum(
                            lax.dot_general(feat.astype(_BF), hW1b[...], _DN,
                                            preferred_element_type=f32)
                            + hb1[...], 0.0)
                        out_ref[...] = lax.dot_general(
                            hh.astype(_BF), hW2b[...], _DN,
                            preferred_element_type=f32) + hb2[...]
                else:
                    out_ref[0] = e


def _const(*_):
    return (0, 0)


def _adj_map(gg, q, r):
    return (jnp.minimum(gg, B - 1), jnp.where(gg < B, r, R - 1), 0)


def _stream_call(adj, W1, b1, W2, b2, head_args=None):
    f32 = jnp.float32
    head = head_args is not None
    in_specs = [
        pl.BlockSpec((1, TM, N), _adj_map),
        pl.BlockSpec((N, HID), _const),
        pl.BlockSpec((1, HID), _const),
        pl.BlockSpec((HID, EMB), _const),
        pl.BlockSpec((1, EMB), _const),
    ]
    scratch = [
        pltpu.VMEM((2, N, N), _BF),
        pltpu.VMEM((R, N, TM), _BF),
        pltpu.VMEM((R, N, TM), f32),
        pltpu.VMEM((N, HID), f32),
        pltpu.VMEM((1, N), f32),
        pltpu.VMEM((2, N, 1), f32),
    ]
    operands = [adj, W1.astype(_BF), b1, W2.astype(_BF), b2]
    if head:
        in_specs += [
            pl.BlockSpec((B, EMB), _const),
            pl.BlockSpec((2 * EMB, HID), _const),
            pl.BlockSpec((1, HID), _const),
            pl.BlockSpec((HID, NC), _const),
            pl.BlockSpec((1, NC), _const),
        ]
        scratch.append(pltpu.VMEM((B, EMB), f32))
        fe, hW1, hb1, hW2, hb2 = head_args
        operands += [fe, hW1.astype(_BF), hb1, hW2.astype(_BF), hb2]
        out_spec = pl.BlockSpec((B, NC), _const)
        out_shape = jax.ShapeDtypeStruct((B, NC), f32)
    else:
        out_spec = pl.BlockSpec(
            (1, 1, EMB), lambda gg, q, r: (jnp.maximum(gg, 1) - 1, 0, 0))
        out_shape = jax.ShapeDtypeStruct((B, 1, EMB), f32)

    return pl.pallas_call(
        functools.partial(_stream_body, head),
        grid=(B + 1, 2, R),
        in_specs=in_specs,
        out_specs=out_spec,
        out_shape=out_shape,
        scratch_shapes=scratch,
        compiler_params=pltpu.CompilerParams(
            dimension_semantics=("arbitrary", "arbitrary", "arbitrary")),
    )(*operands)


@jax.jit
def kernel(fc_adj, sc_adj, fc_W1, fc_b1, fc_W2, fc_b2,
           sc_W1, sc_b1, sc_W2, sc_b2, h_W1, h_b1, h_W2, h_b2):
    femb = _stream_call(fc_adj, fc_W1, fc_b1.reshape(1, HID),
                        fc_W2, fc_b2.reshape(1, EMB)).reshape(B, EMB)
    out = _stream_call(sc_adj, sc_W1, sc_b1.reshape(1, HID),
                       sc_W2, sc_b2.reshape(1, EMB),
                       head_args=(femb, h_W1, h_b1.reshape(1, HID),
                                  h_W2, h_b2.reshape(1, NC)))
    return out


# speedup vs baseline: 1.3956x; 1.2585x over previous
import functools

import jax
import jax.numpy as jnp
from jax import lax
from jax.experimental import pallas as pl
from jax.experimental.pallas import tpu as pltpu

B = 4
N = 1024
HID = 256
EMB = 128
NC = 2
TM = 512
R = N // TM

_DN = (((1,), (0,)), ((), ()))
_DT = (((0,), (0,)), ((), ()))
_BF = jnp.bfloat16


def _col_of(row_vec, c0):
    seg = row_vec[0:1, c0:c0 + TM]
    m = jnp.broadcast_to(seg, (TM, TM))
    ii = lax.broadcasted_iota(jnp.int32, (TM, TM), 0)
    jj = lax.broadcasted_iota(jnp.int32, (TM, TM), 1)
    return jnp.sum(jnp.where(ii == jj, m, 0.0), axis=1, keepdims=True)


def _stream_body(head, *refs):
    if head:
        (A, W1b, b1, W2b, b2, femb, hW1b, hb1, hW2b, hb2, out_ref,
         ab2, t3, S3, S2, deg, dcol2, emb_s) = refs
    else:
        (A, W1b, b1, W2b, b2, out_ref,
         ab2, t3, S3, S2, deg, dcol2) = refs
    gg = pl.program_id(0)
    q = pl.program_id(1)
    r = pl.program_id(2)
    rbase = r * TM
    f32 = jnp.float32
    cur = gg % 2
    prv = 1 - cur

    @pl.when(q == 0)
    def _q0():
        @pl.when(gg < B)
        def _prep():
            a = A[0]
            w = jnp.where(a > 1e-6, a, 0.0)
            ii = lax.broadcasted_iota(jnp.int32, (TM, N), 0) + rbase
            jj = lax.broadcasted_iota(jnp.int32, (TM, N), 1)
            a_hat = jnp.where(ii == jj, 1.0, w)
            colsum = jnp.sum(a_hat, axis=0, keepdims=True)
            deg[...] = jnp.where(r == 0, colsum, deg[...] + colsum)
            ab2[pl.ds(cur, 1), pl.ds(rbase, TM), :] = a_hat.astype(_BF)[None]

        @pl.when(gg >= 1)
        def _s_dot():
            abp = ab2[pl.ds(prv, 1)][0]
            tj = t3[r]
            S3[r] = lax.dot_general(abp, tj, _DT,
                                    preferred_element_type=f32)

    @pl.when(q == 1)
    def _q1():
        @pl.when(gg < B)
        def _tbuild():
            @pl.when(r == 0)
            def _():
                for ci in range(R):
                    dcol2[pl.ds(cur, 1), ci * TM:(ci + 1) * TM, :] = (
                        lax.rsqrt(_col_of(deg[...], ci * TM))[None])

            a = A[0]
            dc = dcol2[pl.ds(cur, 1), pl.ds(rbase, TM), :][0]
            tt = (dc * a).astype(_BF)
            for j in range(R):
                t3[j, pl.ds(rbase, TM), :] = tt[:, j * TM:(j + 1) * TM]

        @pl.when(gg >= 1)
        def _consume():
            dp = dcol2[pl.ds(prv, 1), pl.ds(rbase, TM), :][0]
            S_rt = jnp.concatenate(
                [S3[j, pl.ds(rbase, TM), :] for j in range(R)], axis=1)
            Z = (dp * S_rt).astype(_BF)
            M = lax.dot_general(Z, W1b[...], _DN, preferred_element_type=f32)
            h = jnp.maximum(M + b1[...], 0.0)
            t2v = (dp * h).astype(_BF)
            abp_rt = ab2[pl.ds(prv, 1), pl.ds(rbase, TM), :][0]
            c2 = lax.dot_general(abp_rt, t2v, _DT,
                                 preferred_element_type=f32)
            S2[...] = jnp.where(r == 0, c2, S2[...] + c2)

            @pl.when(r == R - 1)
            def _epilogue():
                acc = jnp.zeros((1, EMB), f32)
                for ci in range(R):
                    sl = slice(ci * TM, (ci + 1) * TM)
                    dpc = dcol2[pl.ds(prv, 1), sl, :][0]
                    Z2 = (dpc * S2[sl, :]).astype(_BF)
                    h2 = lax.dot_general(Z2, W2b[...], _DN,
                                         preferred_element_type=f32) + b2[...]
                    acc = acc + jnp.sum(h2, axis=0, keepdims=True)
                e = acc * (1.0 / N)
                if head:
                    emb_s[pl.ds(jnp.maximum(gg, 1) - 1, 1), :] = e

                    @pl.when(gg == B)
                    def _head():
                        feat = jnp.concatenate([femb[...], emb_s[...]], axis=1)
                        hh = jnp.maximum(
                            lax.dot_general(feat.astype(_BF), hW1b[...], _DN,
                                            preferred_element_type=f32)
                            + hb1[...], 0.0)
                        out_ref[...] = lax.dot_general(
                            hh.astype(_BF), hW2b[...], _DN,
                            preferred_element_type=f32) + hb2[...]
                else:
                    out_ref[0] = e


def _const(*_):
    return (0, 0)


def _adj_map(gg, q, r):
    return (jnp.minimum(gg, B - 1), jnp.where(gg < B, r, R - 1), 0)


def _stream_call(adj, W1, b1, W2, b2, head_args=None):
    f32 = jnp.float32
    head = head_args is not None
    in_specs = [
        pl.BlockSpec((1, TM, N), _adj_map),
        pl.BlockSpec((N, HID), _const),
        pl.BlockSpec((1, HID), _const),
        pl.BlockSpec((HID, EMB), _const),
        pl.BlockSpec((1, EMB), _const),
    ]
    scratch = [
        pltpu.VMEM((2, N, N), _BF),
        pltpu.VMEM((R, N, TM), _BF),
        pltpu.VMEM((R, N, TM), f32),
        pltpu.VMEM((N, HID), f32),
        pltpu.VMEM((1, N), f32),
        pltpu.VMEM((2, N, 1), f32),
    ]
    operands = [adj, W1.astype(_BF), b1, W2.astype(_BF), b2]
    if head:
        in_specs += [
            pl.BlockSpec((B, EMB), _const),
            pl.BlockSpec((2 * EMB, HID), _const),
            pl.BlockSpec((1, HID), _const),
            pl.BlockSpec((HID, NC), _const),
            pl.BlockSpec((1, NC), _const),
        ]
        scratch.append(pltpu.VMEM((B, EMB), f32))
        fe, hW1, hb1, hW2, hb2 = head_args
        operands += [fe, hW1.astype(_BF), hb1, hW2.astype(_BF), hb2]
        out_spec = pl.BlockSpec((B, NC), _const)
        out_shape = jax.ShapeDtypeStruct((B, NC), f32)
    else:
        out_spec = pl.BlockSpec(
            (1, 1, EMB), lambda gg, q, r: (jnp.maximum(gg, 1) - 1, 0, 0))
        out_shape = jax.ShapeDtypeStruct((B, 1, EMB), f32)

    return pl.pallas_call(
        functools.partial(_stream_body, head),
        grid=(B + 1, 2, R),
        in_specs=in_specs,
        out_specs=out_spec,
        out_shape=out_shape,
        scratch_shapes=scratch,
        compiler_params=pltpu.CompilerParams(
            dimension_semantics=("arbitrary", "arbitrary", "arbitrary")),
    )(*operands)


@jax.jit
def kernel(fc_adj, sc_adj, fc_W1, fc_b1, fc_W2, fc_b2,
           sc_W1, sc_b1, sc_W2, sc_b2, h_W1, h_b1, h_W2, h_b2):
    femb = _stream_call(fc_adj, fc_W1, fc_b1.reshape(1, HID),
                        fc_W2, fc_b2.reshape(1, EMB)).reshape(B, EMB)
    out = _stream_call(sc_adj, sc_W1, sc_b1.reshape(1, HID),
                       sc_W2, sc_b2.reshape(1, EMB),
                       head_args=(femb, h_W1, h_b1.reshape(1, HID),
                                  h_W2, h_b2.reshape(1, NC)))
    return out
